# Initial kernel scaffold; baseline (speedup 1.0000x reference)
import jax
import jax.numpy as jnp
from jax.experimental import pallas as pl


def kernel(x, capacity, W):
    raise NotImplementedError("write your pallas kernel here")


# speedup vs baseline: 1.3429x; 1.3429x over previous
import functools

import jax
import jax.numpy as jnp
from jax.experimental import pallas as pl
from jax.experimental.pallas import tpu as pltpu

B, T, D = 2, 4096, 4096
_TT = 1024


def _score_body(x_ref, w_ref, s_ref, p_ref):
    xb = x_ref[0]
    w = w_ref[...]
    s = jax.lax.dot_general(
        xb, w,
        (((1,), (0,)), ((), ())),
        precision=jax.lax.Precision.DEFAULT,
        preferred_element_type=jnp.float32,
    )
    s_ref[0] = s
    p_ref[0] = jax.nn.sigmoid(s)


def _scores_probs(x, w):
    nt = (B * T) // _TT
    x3 = x.reshape(nt, _TT, D)
    outs = pl.pallas_call(
        _score_body,
        grid=(nt,),
        in_specs=[
            pl.BlockSpec((1, _TT, D), lambda i: (i, 0, 0)),
            pl.BlockSpec((D, 1), lambda i: (0, 0)),
        ],
        out_specs=[
            pl.BlockSpec((1, _TT, 1), lambda i: (i, 0, 0)),
            pl.BlockSpec((1, _TT, 1), lambda i: (i, 0, 0)),
        ],
        out_shape=[
            jax.ShapeDtypeStruct((nt, _TT, 1), jnp.float32),
            jax.ShapeDtypeStruct((nt, _TT, 1), jnp.float32),
        ],
        compiler_params=pltpu.CompilerParams(
            vmem_limit_bytes=100 * 1024 * 1024),
    )(x3, w.reshape(D, 1))
    return outs[0].reshape(B, T), outs[1].reshape(B, T)


def _mask_body(cap_ref, p_ref, m_ref):
    cap = cap_ref[0, 0]
    p = p_ref[...]
    keys = jax.lax.bitcast_convert_type(p, jnp.int32)
    capv = jnp.full((B, 1), cap, jnp.int32)

    def bis_step(_, lohi):
        lo, hi = lohi
        mid = lo + ((hi - lo + 1) >> 1)
        cnt = jnp.sum((keys >= mid).astype(jnp.int32), axis=-1, keepdims=True)
        ok = cnt >= capv
        return jnp.where(ok, mid, lo), jnp.where(ok, hi, mid - 1)

    lo0 = jnp.zeros((B, 1), jnp.int32)
    hi0 = jnp.full((B, 1), 0x3F800000, jnp.int32)
    kstar, _ = jax.lax.fori_loop(0, 30, bis_step, (lo0, hi0))

    gt = keys > kstar
    eq = keys == kstar
    g = jnp.sum(gt.astype(jnp.int32), axis=-1, keepdims=True)
    rem = capv - g
    eqi = eq.astype(jnp.int32)
    iota = jax.lax.broadcasted_iota(jnp.int32, (B, T), 1)

    def idx_step(_, lohi):
        lo, hi = lohi
        mid = lo + ((hi - lo + 1) >> 1)
        cnt = jnp.sum(jnp.where(iota < mid, eqi, 0), axis=-1, keepdims=True)
        ok = cnt <= rem
        return jnp.where(ok, mid, lo), jnp.where(ok, hi, mid - 1)

    lo0i = jnp.zeros((B, 1), jnp.int32)
    hi0i = jnp.full((B, 1), T, jnp.int32)
    cstar, _ = jax.lax.fori_loop(0, 13, idx_step, (lo0i, hi0i))

    m_ref[...] = (gt | (eq & (iota < cstar))).astype(jnp.float32)


def _mask(probs, cap2d):
    return pl.pallas_call(
        _mask_body,
        in_specs=[
            pl.BlockSpec((1, 1), lambda: (0, 0)),
            pl.BlockSpec((B, T), lambda: (0, 0)),
        ],
        out_specs=pl.BlockSpec((B, T), lambda: (0, 0)),
        out_shape=jax.ShapeDtypeStruct((B, T), jnp.float32),
    )(cap2d, probs)


def _zloss_body(s_ref, z_ref):
    s = s_ref[...]
    m = jnp.max(s, axis=-1, keepdims=True)
    lse = m + jnp.log(jnp.sum(jnp.exp(s - m), axis=-1, keepdims=True))
    z_ref[...] = jnp.mean(lse * lse).reshape(1, 1)


def _zloss(scores):
    return pl.pallas_call(
        _zloss_body,
        in_specs=[pl.BlockSpec((B, T), lambda: (0, 0))],
        out_specs=pl.BlockSpec((1, 1), lambda: (0, 0)),
        out_shape=jax.ShapeDtypeStruct((1, 1), jnp.float32),
    )(scores)


def kernel(x, capacity, W):
    scores, probs = _scores_probs(x, W)
    cap2d = jnp.asarray(capacity, jnp.int32).reshape(1, 1)
    mask = _mask(probs, cap2d)
    z = _zloss(scores)
    return (mask, probs, z[0, 0])
